# baseline (speedup 1.0000x reference)
import jax
import jax.numpy as jnp
from jax import lax
from jax.experimental import pallas as pl
from jax.experimental.pallas import tpu as pltpu
from jax.experimental.pallas import tpu_sc as plsc

_EMBED = 4096
_ROWS = 1024
_L = 20
_NC = 2
_NS = 16
_NW = _NC * _NS
_RPW = _ROWS // _NW
_C = 8
_Q = _RPW // _C
_NCHUNK = _L * _Q
_NBUF = 3
_MAIN = _NCHUNK - (_NCHUNK % _NBUF)


def _emb_body(table_hbm, x_hbm, out_hbm, xv, idx_v,
              buf0, buf1, buf2, g0, g1, g2, o0, o1, o2):
    bufs = (buf0, buf1, buf2)
    gsems = (g0, g1, g2)
    osems = (o0, o1, o2)

    wid = lax.axis_index("s") * _NC + lax.axis_index("c")
    row_base = wid * _RPW
    pltpu.sync_copy(x_hbm.at[pl.ds(row_base, _RPW)], xv)

    lane = lax.iota(jnp.int32, 16)
    hi = lax.shift_right_logical(lane, 3)
    lo = lax.bitwise_and(lane, 7)
    for m in range(_NCHUNK // 2):
        c0 = 2 * m
        j = c0 // _Q
        q0 = c0 % _Q
        rows = q0 * _C + lane
        cols = jnp.full((16,), j, jnp.int32)
        v = plsc.load_gather(xv, [rows, cols])
        plsc.store_scatter(idx_v, [c0 + hi, lo], v)

    def out_slice(c):
        i0 = row_base + (c % _Q) * _C
        col0 = (c // _Q) * _EMBED
        return out_hbm.at[pl.ds(i0, _C), pl.ds(col0, _EMBED)]

    def gather_start(c, b):
        pltpu.async_copy(table_hbm.at[idx_v.at[c]], bufs[b], gsems[b])

    def gather_wait(c, b):
        pltpu.make_async_copy(table_hbm.at[idx_v.at[c]], bufs[b], gsems[b]).wait()

    def out_start(c, b):
        pltpu.async_copy(bufs[b], out_slice(c), osems[b])

    def out_wait(c, b):
        pltpu.make_async_copy(bufs[b], out_slice(c), osems[b]).wait()

    gather_start(0, 0)
    gather_start(1, 1)

    def step(c, b):
        gather_wait(c, b)
        out_start(c, b)
        b2 = (b + 2) % _NBUF

        @pl.when(c + 2 < _NCHUNK)
        def _():
            @pl.when(c >= 1)
            def _():
                out_wait(c - 1, b2)

            gather_start(c + 2, b2)

    @pl.loop(0, _MAIN, step=_NBUF)
    def _(g):
        for b in range(_NBUF):
            step(g + b, b)

    for c in range(_MAIN, _NCHUNK):
        step(c, c % _NBUF)

    for c in range(_NCHUNK - _NBUF, _NCHUNK):
        out_wait(c, c % _NBUF)


@jax.jit
def kernel(x, table):
    mesh = plsc.VectorSubcoreMesh(core_axis_name="c", subcore_axis_name="s")
    return pl.kernel(
        _emb_body,
        out_type=jax.ShapeDtypeStruct((_ROWS, _L * _EMBED), jnp.float32),
        mesh=mesh,
        compiler_params=pltpu.CompilerParams(needs_layout_passes=False),
        scratch_types=[
            pltpu.VMEM((_RPW, _L), jnp.int32),
            pltpu.VMEM((_NCHUNK, _C), jnp.int32),
            pltpu.VMEM((_C, _EMBED), jnp.float32),
            pltpu.VMEM((_C, _EMBED), jnp.float32),
            pltpu.VMEM((_C, _EMBED), jnp.float32),
            pltpu.SemaphoreType.DMA,
            pltpu.SemaphoreType.DMA,
            pltpu.SemaphoreType.DMA,
            pltpu.SemaphoreType.DMA,
            pltpu.SemaphoreType.DMA,
            pltpu.SemaphoreType.DMA,
        ],
    )(table, x.astype(jnp.int32))
